# Initial kernel scaffold; baseline (speedup 1.0000x reference)
import jax
import jax.numpy as jnp
from jax.experimental import pallas as pl


def kernel(input_tensor, embedding_table, proj_W, proj_b):
    raise NotImplementedError("write your pallas kernel here")


# speedup vs baseline: 1.8349x; 1.8349x over previous
import functools

import jax
import jax.numpy as jnp
from jax import lax
from jax.experimental import pallas as pl
from jax.experimental.pallas import tpu as pltpu
from jax.experimental.pallas import tpu_sc as plsc

_VOCAB = 32000
_D = 2048
_NTOK = 4 * 2048

_NC, _NS = 2, 16
_NW = _NC * _NS
_B_PER_W = _NTOK // _NW
_CH = 16
_NCHUNK = _B_PER_W // _CH


def _sc_gather(table, idx):
    mesh = plsc.VectorSubcoreMesh(core_axis_name="c", subcore_axis_name="s")

    @functools.partial(
        pl.kernel,
        mesh=mesh,
        out_type=jax.ShapeDtypeStruct((_NTOK, _D), jnp.float32),
        scratch_types=[
            pltpu.VMEM((_B_PER_W,), jnp.int32),
            pltpu.VMEM((_CH, _D), jnp.float32),
            pltpu.SemaphoreType.DMA,
        ],
    )
    def gather_kernel(table_hbm, idx_hbm, out_hbm, idx_v, rows_v, sem):
        wid = lax.axis_index("s") * _NC + lax.axis_index("c")
        base = wid * _B_PER_W
        pltpu.sync_copy(idx_hbm.at[pl.ds(base, _B_PER_W)], idx_v)

        @pl.loop(0, _NCHUNK)
        def _(j):
            off = j * _CH
            pltpu.async_copy(
                table_hbm.at[idx_v.at[pl.ds(off, _CH)]], rows_v, sem
            ).wait()
            pltpu.sync_copy(rows_v, out_hbm.at[pl.ds(base + off, _CH)])

    return gather_kernel(table, idx)


def _tc_project(x, wt, bias):
    bm = 512

    def mm_kernel(x_ref, wt_ref, b_ref, o_ref):
        xb = x_ref[...].astype(jnp.bfloat16)
        acc = jnp.dot(xb, wt_ref[...], preferred_element_type=jnp.float32)
        o_ref[...] = acc + b_ref[...]

    return pl.pallas_call(
        mm_kernel,
        grid=(_NTOK // bm,),
        in_specs=[
            pl.BlockSpec((bm, _D), lambda i: (i, 0)),
            pl.BlockSpec((_D, _D), lambda i: (0, 0)),
            pl.BlockSpec((1, _D), lambda i: (0, 0)),
        ],
        out_specs=pl.BlockSpec((bm, _D), lambda i: (i, 0)),
        out_shape=jax.ShapeDtypeStruct((_NTOK, _D), jnp.float32),
    )(x, wt, bias)


def kernel(input_tensor, embedding_table, proj_W, proj_b):
    b, s = input_tensor.shape
    idx = input_tensor.reshape(-1).astype(jnp.int32)
    gathered = _sc_gather(embedding_table, idx)
    wt = proj_W.T.astype(jnp.bfloat16)
    out = _tc_project(gathered, wt, proj_b.reshape(1, -1))
    return out.reshape(b, s, -1)
